# Initial kernel scaffold; baseline (speedup 1.0000x reference)
import jax
import jax.numpy as jnp
from jax.experimental import pallas as pl


def kernel(query_signature, k, gallery_signatures):
    raise NotImplementedError("write your pallas kernel here")


# speedup vs baseline: 1.4296x; 1.4296x over previous
import jax
import jax.numpy as jnp
from jax.experimental import pallas as pl
from jax.experimental.pallas import tpu as pltpu

B = 8
D = 192
N = 200000
K = 16
C = 8192
G = -(-N // C)

_NEG = float("-inf")
_IMAX = 2**31 - 1


def _topk_kernel(q_ref, g_ref, idx_out, val_out, bs_ref, bi_ref):
    i = pl.program_id(0)

    @pl.when(i == 0)
    def _init():
        bs_ref[...] = jnp.full((B, K), _NEG, jnp.float32)
        bi_ref[...] = jnp.zeros((B, K), jnp.int32)

    s = jax.lax.dot_general(
        q_ref[...], g_ref[...],
        dimension_numbers=(((1,), (1,)), ((), ())),
        preferred_element_type=jnp.float32,
    )
    col = i * C + jax.lax.broadcasted_iota(jnp.int32, (B, C), 1)
    s = jnp.where(col < N, s, _NEG)

    cs = bs_ref[...]
    ci = bi_ref[...]
    new_s = []
    new_i = []
    for _ in range(K):
        m = jnp.maximum(
            jnp.max(s, axis=1, keepdims=True),
            jnp.max(cs, axis=1, keepdims=True),
        )
        am = jnp.minimum(
            jnp.min(jnp.where(s == m, col, _IMAX), axis=1, keepdims=True),
            jnp.min(jnp.where(cs == m, ci, _IMAX), axis=1, keepdims=True),
        )
        new_s.append(m)
        new_i.append(am)
        s = jnp.where((s == m) & (col == am), _NEG, s)
        cs = jnp.where((cs == m) & (ci == am), _NEG, cs)
    bs_ref[...] = jnp.concatenate(new_s, axis=1)
    bi_ref[...] = jnp.concatenate(new_i, axis=1)

    @pl.when(i == G - 1)
    def _fin():
        idx_out[...] = bi_ref[...]
        val_out[...] = bs_ref[...]


def kernel(query_signature, k, gallery_signatures):
    del k
    idx, val = pl.pallas_call(
        _topk_kernel,
        grid=(G,),
        in_specs=[
            pl.BlockSpec((B, D), lambda i: (0, 0)),
            pl.BlockSpec((C, D), lambda i: (i, 0)),
        ],
        out_specs=[
            pl.BlockSpec((B, K), lambda i: (0, 0)),
            pl.BlockSpec((B, K), lambda i: (0, 0)),
        ],
        out_shape=[
            jax.ShapeDtypeStruct((B, K), jnp.int32),
            jax.ShapeDtypeStruct((B, K), jnp.float32),
        ],
        scratch_shapes=[
            pltpu.VMEM((B, K), jnp.float32),
            pltpu.VMEM((B, K), jnp.int32),
        ],
    )(query_signature, gallery_signatures)
    return idx, val
